# Initial kernel scaffold; baseline (speedup 1.0000x reference)
import jax
import jax.numpy as jnp
from jax.experimental import pallas as pl


def kernel(x, table):
    raise NotImplementedError("write your pallas kernel here")


# speedup vs baseline: 2.9603x; 2.9603x over previous
import functools

import jax
import jax.numpy as jnp
from jax import lax
from jax.experimental import pallas as pl
from jax.experimental.pallas import tpu as pltpu
from jax.experimental.pallas import tpu_sc as plsc

B_TOK = 4096 * 50
D = 128
NUM_CORES = 2
NUM_SUBCORES = 16
NW = NUM_CORES * NUM_SUBCORES
BPW = B_TOK // NW
CHUNK = 128
NCHUNK = BPW // CHUNK


@functools.partial(
    pl.kernel,
    out_type=jax.ShapeDtypeStruct((B_TOK, D), jnp.float32),
    mesh=plsc.VectorSubcoreMesh(core_axis_name="c", subcore_axis_name="s"),
    scratch_types=[
        pltpu.VMEM((NCHUNK, CHUNK), jnp.int32),
        pltpu.VMEM((CHUNK, D), jnp.float32),
        pltpu.SemaphoreType.DMA,
    ],
)
def _embed_gather(idx_hbm, table_hbm, out_hbm, idx_v, rows_v, gsem):
    wid = lax.axis_index("s") * NUM_CORES + lax.axis_index("c")
    pltpu.sync_copy(idx_hbm.at[wid], idx_v)
    base = pl.multiple_of(wid * BPW, CHUNK)

    def body(j, carry):
        pltpu.async_copy(table_hbm.at[idx_v.at[j]], rows_v, gsem).wait()
        off = pl.multiple_of(base + j * CHUNK, CHUNK)
        pltpu.sync_copy(rows_v, out_hbm.at[pl.ds(off, CHUNK)])
        return carry

    lax.fori_loop(0, NCHUNK, body, 0)


def kernel(x, table):
    idx = x.reshape(NW, NCHUNK, CHUNK).astype(jnp.int32)
    out = _embed_gather(idx, table)
    return out.reshape(x.shape[0], x.shape[1], D)


# speedup vs baseline: 3.3035x; 1.1159x over previous
import functools

import jax
import jax.numpy as jnp
from jax import lax
from jax.experimental import pallas as pl
from jax.experimental.pallas import tpu as pltpu
from jax.experimental.pallas import tpu_sc as plsc

B_TOK = 4096 * 50
D = 128
NUM_CORES = 2
NUM_SUBCORES = 16
NW = NUM_CORES * NUM_SUBCORES
BPW = B_TOK // NW
CHUNK = 128
NCHUNK = BPW // CHUNK
NBUF = 5
NOUTER = NCHUNK // NBUF


@functools.partial(
    pl.kernel,
    out_type=jax.ShapeDtypeStruct((B_TOK, D), jnp.float32),
    mesh=plsc.VectorSubcoreMesh(core_axis_name="c", subcore_axis_name="s"),
    scratch_types=[
        pltpu.VMEM((NCHUNK, CHUNK), jnp.int32),
        pltpu.VMEM((NBUF, CHUNK, D), jnp.float32),
    ]
    + [pltpu.SemaphoreType.DMA] * (2 * NBUF),
)
def _embed_gather(idx_hbm, table_hbm, out_hbm, idx_v, rows_v, *sems):
    gsems = sems[:NBUF]
    osems = sems[NBUF:]
    wid = lax.axis_index("s") * NUM_CORES + lax.axis_index("c")
    pltpu.sync_copy(idx_hbm.at[wid], idx_v)
    base = pl.multiple_of(wid * BPW, CHUNK)

    def out_at(j):
        off = pl.multiple_of(base + j * CHUNK, CHUNK)
        return out_hbm.at[pl.ds(off, CHUNK)]

    def g_copy(j, b):
        return pltpu.make_async_copy(
            table_hbm.at[idx_v.at[j]], rows_v.at[b], gsems[b])

    def w_copy(j, b):
        return pltpu.make_async_copy(rows_v.at[b], out_at(j), osems[b])

    for b in range(NBUF):
        g_copy(b, b).start()

    def body(i, carry):
        for b in range(NBUF):
            j = i * NBUF + b
            g_copy(j, b).wait()
            w_copy(j, b).start()
        for b in range(NBUF):
            j = i * NBUF + b
            w_copy(j, b).wait()
            g_copy(j + NBUF, b).start()
        return carry

    lax.fori_loop(0, NOUTER - 1, body, 0)

    for b in range(NBUF):
        j = (NOUTER - 1) * NBUF + b
        g_copy(j, b).wait()
        w_copy(j, b).start()
    for b in range(NBUF):
        j = (NOUTER - 1) * NBUF + b
        w_copy(j, b).wait()


def kernel(x, table):
    idx = x.reshape(NW, NCHUNK, CHUNK).astype(jnp.int32)
    out = _embed_gather(idx, table)
    return out.reshape(x.shape[0], x.shape[1], D)


# speedup vs baseline: 5.8684x; 1.7764x over previous
import functools

import jax
import jax.numpy as jnp
from jax import lax
from jax.experimental import pallas as pl
from jax.experimental.pallas import tpu as pltpu
from jax.experimental.pallas import tpu_sc as plsc

B = 4096
S = 50
D = 128
NUM_CORES = 2
NUM_SUBCORES = 16
NW = NUM_CORES * NUM_SUBCORES
RPW = B // NW
G = 4
NGRP = RPW // G
NBUF = 4
NOUTER = NGRP // NBUF


@functools.partial(
    pl.kernel,
    out_type=jax.ShapeDtypeStruct((B, S, D), jnp.float32),
    mesh=plsc.VectorSubcoreMesh(core_axis_name="c", subcore_axis_name="s"),
    scratch_types=[
        pltpu.VMEM((RPW, S), jnp.int32),
        pltpu.VMEM((NBUF, G, S, D), jnp.float32),
    ]
    + [pltpu.SemaphoreType.DMA] * (2 * NBUF),
)
def _embed_gather(x_hbm, table_hbm, out_hbm, idx_v, rows_v, *sems):
    gsems = sems[:NBUF]
    osems = sems[NBUF:]
    wid = lax.axis_index("s") * NUM_CORES + lax.axis_index("c")
    base = pl.multiple_of(wid * RPW, RPW)
    pltpu.sync_copy(x_hbm.at[pl.ds(base, RPW)], idx_v)

    def g_copy(grp, g, b):
        return pltpu.make_async_copy(
            table_hbm.at[idx_v.at[grp * G + g]],
            rows_v.at[b, g],
            gsems[b],
        )

    def w_copy(grp, b):
        row = base + grp * G
        return pltpu.make_async_copy(
            rows_v.at[b],
            out_hbm.at[pl.ds(row, G)],
            osems[b],
        )

    for b in range(NBUF):
        for g in range(G):
            g_copy(b, g, b).start()

    def body(i, carry):
        for b in range(NBUF):
            grp = i * NBUF + b
            for g in range(G):
                g_copy(grp, g, b).wait()
            w_copy(grp, b).start()
        for b in range(NBUF):
            grp = i * NBUF + b
            w_copy(grp, b).wait()
            for g in range(G):
                g_copy(grp + NBUF, g, b).start()
        return carry

    lax.fori_loop(0, NOUTER - 1, body, 0)

    for b in range(NBUF):
        grp = (NOUTER - 1) * NBUF + b
        for g in range(G):
            g_copy(grp, g, b).wait()
        w_copy(grp, b).start()
    for b in range(NBUF):
        grp = (NOUTER - 1) * NBUF + b
        w_copy(grp, b).wait()


def kernel(x, table):
    return _embed_gather(x.astype(jnp.int32), table)
